# baseline (speedup 1.0000x reference)
import functools

import jax
import jax.numpy as jnp
from jax import lax
from jax.experimental import pallas as pl
from jax.experimental.pallas import tpu as pltpu
from jax.experimental.pallas import tpu_sc as plsc

_B = 16384
_NC = 2
_NS = 16
_NW = _NC * _NS
_G = 4 * _B
_RPW = _G // _NW
_CH = 256
_NCH = _RPW // _CH

_FN = 32
_MF = 8

_sc_mesh = plsc.VectorSubcoreMesh(core_axis_name="c", subcore_axis_name="s")


@functools.partial(
    pl.kernel,
    out_type=jax.ShapeDtypeStruct((_G, 128), jnp.float32),
    mesh=_sc_mesh,
    scratch_types=(
        pltpu.VMEM((_RPW,), jnp.int32),
        pltpu.VMEM((_CH, 128), jnp.float32),
        pltpu.VMEM((_CH, 128), jnp.float32),
        pltpu.SemaphoreType.DMA,
        pltpu.SemaphoreType.DMA,
        pltpu.SemaphoreType.DMA,
        pltpu.SemaphoreType.DMA,
    ),
)
def _sc_gather(idx_hbm, tab, out, idx_v, v0, v1, g0, g1, o0, o1):
    wid = lax.axis_index("s") * _NC + lax.axis_index("c")
    base = wid * _RPW
    pltpu.sync_copy(idx_hbm.at[pl.ds(base, _RPW)], idx_v)
    bufs = (v0, v1)
    gsems = (g0, g1)
    osems = (o0, o1)
    gcp = [None] * _NCH
    ocp = [None] * _NCH
    for c in range(_NCH):
        if c >= 2:
            ocp[c - 2].wait()
        gcp[c] = pltpu.async_copy(
            tab.at[idx_v.at[pl.ds(c * _CH, _CH)]], bufs[c % 2], gsems[c % 2])
        if c >= 1:
            gcp[c - 1].wait()
            ocp[c - 1] = pltpu.async_copy(
                bufs[(c - 1) % 2], out.at[pl.ds(base + (c - 1) * _CH, _CH)],
                osems[(c - 1) % 2])
    gcp[_NCH - 1].wait()
    ocp[_NCH - 1] = pltpu.async_copy(
        bufs[(_NCH - 1) % 2], out.at[pl.ds(base + (_NCH - 1) * _CH, _CH)],
        osems[(_NCH - 1) % 2])
    ocp[_NCH - 2].wait()
    ocp[_NCH - 1].wait()


def _mlp_body(gfu_ref, gfi_ref, gmu_ref, gmi_ref, u_ref, i_ref,
              w1u_ref, w1i_ref, b1_ref, w2_ref, b2_ref, w3_ref, b3_ref,
              womf_ref, woh_ref, bo_ref, out_ref):
    f32 = jnp.float32
    u = u_ref[...]
    it = i_ref[...]

    def sel(g_ref, q, width):
        m = None
        for k in range(128 // width):
            part = g_ref[:, k * width:(k + 1) * width]
            mk = (q == k).astype(f32)
            m = part * mk if m is None else m + part * mk
        return m

    fnu = sel(gfu_ref, u & 3, _FN)
    fni = sel(gfi_ref, it & 3, _FN)
    mfu = sel(gmu_ref, u & 15, _MF)
    mfi = sel(gmi_ref, it & 15, _MF)

    h = jnp.dot(fnu, w1u_ref[...], preferred_element_type=f32)
    h += jnp.dot(fni, w1i_ref[...], preferred_element_type=f32)
    h = jnp.maximum(h + b1_ref[...], 0.0)
    h = jnp.maximum(
        jnp.dot(h, w2_ref[...], preferred_element_type=f32) + b2_ref[...], 0.0)
    h = jnp.maximum(
        jnp.dot(h, w3_ref[...], preferred_element_type=f32) + b3_ref[...], 0.0)
    r = jnp.dot(mfu * mfi, womf_ref[...], preferred_element_type=f32)
    r += jnp.dot(h, woh_ref[...], preferred_element_type=f32)
    out_ref[...] = r[:, 0] + bo_ref[0, 0]


def kernel(user, item, mf_emb_user, mf_emb_item, fn_emb_user, fn_emb_item,
           W1, b1, W2, b2, W3, b3, Wo, bo):
    ui = user.astype(jnp.int32)
    it = item.astype(jnp.int32)
    tab = jnp.concatenate([
        fn_emb_user.reshape(25000, 128),
        fn_emb_item.reshape(25000, 128),
        mf_emb_user.reshape(6250, 128),
        mf_emb_item.reshape(6250, 128),
    ], axis=0)
    idx = jnp.concatenate([
        ui >> 2, 25000 + (it >> 2), 50000 + (ui >> 4), 56250 + (it >> 4)])

    g = _sc_gather(idx, tab)

    blk = 2048
    grid = _B // blk

    def _w(shape):
        return pl.BlockSpec(shape, lambda i: (0, 0))

    out = pl.pallas_call(
        _mlp_body,
        grid=(grid,),
        in_specs=[
            pl.BlockSpec((blk, 128), lambda i: (i, 0)),
            pl.BlockSpec((blk, 128), lambda i: (i + 8, 0)),
            pl.BlockSpec((blk, 128), lambda i: (i + 16, 0)),
            pl.BlockSpec((blk, 128), lambda i: (i + 24, 0)),
            pl.BlockSpec((blk, 1), lambda i: (i, 0)),
            pl.BlockSpec((blk, 1), lambda i: (i, 0)),
            _w((_FN, 64)), _w((_FN, 64)), _w((1, 64)),
            _w((64, 32)), _w((1, 32)),
            _w((32, 16)), _w((1, 16)),
            _w((_MF, 1)), _w((16, 1)), _w((1, 1)),
        ],
        out_specs=pl.BlockSpec((blk,), lambda i: (i,)),
        out_shape=jax.ShapeDtypeStruct((_B,), jnp.float32),
    )(g, g, g, g, ui.reshape(_B, 1), it.reshape(_B, 1),
      W1[:_FN], W1[_FN:], b1.reshape(1, 64),
      W2, b2.reshape(1, 32),
      W3, b3.reshape(1, 16),
      Wo[:_MF], Wo[_MF:], bo.reshape(1, 1))
    return out


# speedup vs baseline: 1.3402x; 1.3402x over previous
import functools

import jax
import jax.numpy as jnp
from jax import lax
from jax.experimental import pallas as pl
from jax.experimental.pallas import tpu as pltpu
from jax.experimental.pallas import tpu_sc as plsc

_B = 16384
_NC = 2
_NS = 16
_NW = _NC * _NS
_G = 4 * _B
_RPW = _G // _NW
_CH = 512
_NCH = _RPW // _CH

_FN = 32
_MF = 8

_sc_mesh = plsc.VectorSubcoreMesh(core_axis_name="c", subcore_axis_name="s")


@functools.partial(
    pl.kernel,
    out_type=jax.ShapeDtypeStruct((_G, 128), jnp.float32),
    mesh=_sc_mesh,
    scratch_types=(
        pltpu.VMEM((_RPW,), jnp.int32),
        pltpu.VMEM((_CH, 128), jnp.float32),
        pltpu.SemaphoreType.DMA,
        pltpu.SemaphoreType.DMA,
    ),
)
def _sc_gather(idx_hbm, tab, out, idx_v, buf, gsem, osem):
    wid = lax.axis_index("s") * _NC + lax.axis_index("c")
    base = wid * _RPW
    pltpu.sync_copy(idx_hbm.at[pl.ds(base, _RPW)], idx_v)

    def body(c, carry):
        off = c * _CH
        pltpu.async_copy(
            tab.at[idx_v.at[pl.ds(off, _CH)]], buf, gsem).wait()
        pltpu.async_copy(buf, out.at[pl.ds(base + off, _CH)], osem).wait()
        return carry

    lax.fori_loop(0, _NCH, body, 0)


def _mlp_body(gfu_ref, gfi_ref, gmu_ref, gmi_ref, u_ref, i_ref,
              w1u4_ref, w1i4_ref, b1_ref, w2_ref, b2_ref, w3_ref, b3_ref,
              t8u_ref, t8i_ref, woh_ref, bo_ref, out_ref):
    f32 = jnp.float32
    blk = gfu_ref.shape[0]
    u = u_ref[...]
    it = i_ref[...]
    lane = lax.broadcasted_iota(jnp.int32, (blk, 128), 1)

    gfu = gfu_ref[...] * ((lane >> 5) == (u & 3)).astype(f32)
    gfi = gfi_ref[...] * ((lane >> 5) == (it & 3)).astype(f32)
    gmu = gmu_ref[...] * ((lane >> 3) == (u & 15)).astype(f32)
    gmi = gmi_ref[...] * ((lane >> 3) == (it & 15)).astype(f32)

    h = jnp.dot(gfu, w1u4_ref[...], preferred_element_type=f32)
    h += jnp.dot(gfi, w1i4_ref[...], preferred_element_type=f32)
    h = jnp.maximum(h + b1_ref[...], 0.0)
    h = jnp.maximum(
        jnp.dot(h, w2_ref[...], preferred_element_type=f32) + b2_ref[...], 0.0)
    h = jnp.maximum(
        jnp.dot(h, w3_ref[...], preferred_element_type=f32) + b3_ref[...], 0.0)
    mfu = jnp.dot(gmu, t8u_ref[...], preferred_element_type=f32)
    mfi = jnp.dot(gmi, t8i_ref[...], preferred_element_type=f32)
    r = jnp.sum(mfu * mfi, axis=1, keepdims=True)
    r += jnp.dot(h, woh_ref[...], preferred_element_type=f32)
    out_ref[...] = r[:, 0] + bo_ref[0, 0]


def kernel(user, item, mf_emb_user, mf_emb_item, fn_emb_user, fn_emb_item,
           W1, b1, W2, b2, W3, b3, Wo, bo):
    ui = user.astype(jnp.int32)
    it = item.astype(jnp.int32)
    tab = jnp.concatenate([
        fn_emb_user.reshape(25000, 128),
        fn_emb_item.reshape(25000, 128),
        mf_emb_user.reshape(6250, 128),
        mf_emb_item.reshape(6250, 128),
    ], axis=0)
    idx = jnp.concatenate([
        ui >> 2, 25000 + (it >> 2), 50000 + (ui >> 4), 56250 + (it >> 4)])

    g = _sc_gather(idx, tab)

    eye8 = jnp.eye(8, dtype=jnp.float32)
    t8 = jnp.tile(eye8, (16, 1))
    t8u = t8 * Wo[:_MF, 0][None, :]
    w1u4 = jnp.tile(W1[:_FN], (4, 1))
    w1i4 = jnp.tile(W1[_FN:], (4, 1))

    blk = 2048
    grid = _B // blk

    def _w(shape):
        return pl.BlockSpec(shape, lambda i: (0, 0))

    out = pl.pallas_call(
        _mlp_body,
        grid=(grid,),
        in_specs=[
            pl.BlockSpec((blk, 128), lambda i: (i, 0)),
            pl.BlockSpec((blk, 128), lambda i: (i + 8, 0)),
            pl.BlockSpec((blk, 128), lambda i: (i + 16, 0)),
            pl.BlockSpec((blk, 128), lambda i: (i + 24, 0)),
            pl.BlockSpec((blk, 1), lambda i: (i, 0)),
            pl.BlockSpec((blk, 1), lambda i: (i, 0)),
            _w((128, 64)), _w((128, 64)), _w((1, 64)),
            _w((64, 32)), _w((1, 32)),
            _w((32, 16)), _w((1, 16)),
            _w((128, _MF)), _w((128, _MF)), _w((16, 1)), _w((1, 1)),
        ],
        out_specs=pl.BlockSpec((blk,), lambda i: (i,)),
        out_shape=jax.ShapeDtypeStruct((_B,), jnp.float32),
    )(g, g, g, g, ui.reshape(_B, 1), it.reshape(_B, 1),
      w1u4, w1i4, b1.reshape(1, 64),
      W2, b2.reshape(1, 32),
      W3, b3.reshape(1, 16),
      t8u, t8, Wo[_MF:], bo.reshape(1, 1))
    return out


# speedup vs baseline: 1.6924x; 1.2628x over previous
import functools

import jax
import jax.numpy as jnp
from jax import lax
from jax.experimental import pallas as pl
from jax.experimental.pallas import tpu as pltpu
from jax.experimental.pallas import tpu_sc as plsc

_B = 16384
_NC = 2
_NS = 16
_NW = _NC * _NS
_G = 4 * _B
_RPW = _G // _NW
_CH = 512
_NCH = _RPW // _CH

_FN = 32
_MF = 8

_sc_mesh = plsc.VectorSubcoreMesh(core_axis_name="c", subcore_axis_name="s")


@functools.partial(
    pl.kernel,
    out_type=jax.ShapeDtypeStruct((_G, 128), jnp.float32),
    mesh=_sc_mesh,
    scratch_types=(
        pltpu.VMEM((_RPW,), jnp.int32),
        pltpu.VMEM((_CH, 128), jnp.float32),
        pltpu.SemaphoreType.DMA,
        pltpu.SemaphoreType.DMA,
    ),
)
def _sc_gather(idx_hbm, t0, t1, t2, t3, out, idx_v, buf, gsem, osem):
    wid = lax.axis_index("s") * _NC + lax.axis_index("c")
    base = wid * _RPW
    pltpu.sync_copy(idx_hbm.at[pl.ds(base, _RPW)], idx_v)
    seg = wid >> 3
    for s, tab in enumerate((t0, t1, t2, t3)):
        @pl.when(seg == s)
        def _():
            def body(c, carry):
                off = c * _CH
                pltpu.async_copy(
                    tab.at[idx_v.at[pl.ds(off, _CH)]], buf, gsem).wait()
                pltpu.async_copy(
                    buf, out.at[pl.ds(base + off, _CH)], osem).wait()
                return carry
            lax.fori_loop(0, _NCH, body, 0)


def _mlp_body(gfu_ref, gfi_ref, gmu_ref, gmi_ref, u_ref, i_ref,
              w1u4_ref, w1i4_ref, b1_ref, w2_ref, b2_ref, w3_ref, b3_ref,
              t8u_ref, t8i_ref, woh_ref, bo_ref, out_ref):
    f32 = jnp.float32
    blk = gfu_ref.shape[0]
    u = u_ref[...]
    it = i_ref[...]
    lane = lax.broadcasted_iota(jnp.int32, (blk, 128), 1)

    gfu = gfu_ref[...] * ((lane >> 5) == (u & 3)).astype(f32)
    gfi = gfi_ref[...] * ((lane >> 5) == (it & 3)).astype(f32)
    gmu = gmu_ref[...] * ((lane >> 3) == (u & 15)).astype(f32)
    gmi = gmi_ref[...] * ((lane >> 3) == (it & 15)).astype(f32)

    h = jnp.dot(gfu, w1u4_ref[...], preferred_element_type=f32)
    h += jnp.dot(gfi, w1i4_ref[...], preferred_element_type=f32)
    h = jnp.maximum(h + b1_ref[...], 0.0)
    h = jnp.maximum(
        jnp.dot(h, w2_ref[...], preferred_element_type=f32) + b2_ref[...], 0.0)
    h = jnp.maximum(
        jnp.dot(h, w3_ref[...], preferred_element_type=f32) + b3_ref[...], 0.0)
    mfu = jnp.dot(gmu, t8u_ref[...], preferred_element_type=f32)
    mfi = jnp.dot(gmi, t8i_ref[...], preferred_element_type=f32)
    r = jnp.sum(mfu * mfi, axis=1, keepdims=True)
    r += jnp.dot(h, woh_ref[...], preferred_element_type=f32)
    out_ref[...] = r[:, 0] + bo_ref[0, 0]


def kernel(user, item, mf_emb_user, mf_emb_item, fn_emb_user, fn_emb_item,
           W1, b1, W2, b2, W3, b3, Wo, bo):
    ui = user.astype(jnp.int32)
    it = item.astype(jnp.int32)
    idx = jnp.concatenate([ui >> 2, it >> 2, ui >> 4, it >> 4])

    g = _sc_gather(idx,
                   fn_emb_user.reshape(25000, 128),
                   fn_emb_item.reshape(25000, 128),
                   mf_emb_user.reshape(6250, 128),
                   mf_emb_item.reshape(6250, 128))

    eye8 = jnp.eye(8, dtype=jnp.float32)
    t8 = jnp.tile(eye8, (16, 1))
    t8u = t8 * Wo[:_MF, 0][None, :]
    w1u4 = jnp.tile(W1[:_FN], (4, 1))
    w1i4 = jnp.tile(W1[_FN:], (4, 1))

    blk = 2048
    grid = _B // blk

    def _w(shape):
        return pl.BlockSpec(shape, lambda i: (0, 0))

    out = pl.pallas_call(
        _mlp_body,
        grid=(grid,),
        in_specs=[
            pl.BlockSpec((blk, 128), lambda i: (i, 0)),
            pl.BlockSpec((blk, 128), lambda i: (i + 8, 0)),
            pl.BlockSpec((blk, 128), lambda i: (i + 16, 0)),
            pl.BlockSpec((blk, 128), lambda i: (i + 24, 0)),
            pl.BlockSpec((blk, 1), lambda i: (i, 0)),
            pl.BlockSpec((blk, 1), lambda i: (i, 0)),
            _w((128, 64)), _w((128, 64)), _w((1, 64)),
            _w((64, 32)), _w((1, 32)),
            _w((32, 16)), _w((1, 16)),
            _w((128, _MF)), _w((128, _MF)), _w((16, 1)), _w((1, 1)),
        ],
        out_specs=pl.BlockSpec((blk,), lambda i: (i,)),
        out_shape=jax.ShapeDtypeStruct((_B,), jnp.float32),
    )(g, g, g, g, ui.reshape(_B, 1), it.reshape(_B, 1),
      w1u4, w1i4, b1.reshape(1, 64),
      W2, b2.reshape(1, 32),
      W3, b3.reshape(1, 16),
      t8u, t8, Wo[_MF:], bo.reshape(1, 1))
    return out


# speedup vs baseline: 1.7207x; 1.0167x over previous
import functools

import jax
import jax.numpy as jnp
from jax import lax
from jax.experimental import pallas as pl
from jax.experimental.pallas import tpu as pltpu
from jax.experimental.pallas import tpu_sc as plsc

_B = 16384
_NCHUNK = 2
_CB = _B // _NCHUNK
_NC = 2
_NS = 16
_NW = _NC * _NS
_BPW = _CB // _NW

_FN = 32
_MF = 8

_sc_mesh = plsc.VectorSubcoreMesh(core_axis_name="c", subcore_axis_name="s")


@functools.partial(
    pl.kernel,
    out_type=(
        jax.ShapeDtypeStruct((_CB, _FN), jnp.float32),
        jax.ShapeDtypeStruct((_CB, _FN), jnp.float32),
        jax.ShapeDtypeStruct((_CB, _MF), jnp.float32),
        jax.ShapeDtypeStruct((_CB, _MF), jnp.float32),
    ),
    mesh=_sc_mesh,
    scratch_types=(
        pltpu.VMEM((_BPW,), jnp.int32),
        pltpu.VMEM((_BPW,), jnp.int32),
        pltpu.VMEM((_BPW, _FN), jnp.float32),
        pltpu.VMEM((_BPW, _FN), jnp.float32),
        pltpu.VMEM((_BPW, _MF), jnp.float32),
        pltpu.VMEM((_BPW, _MF), jnp.float32),
        pltpu.SemaphoreType.DMA,
        pltpu.SemaphoreType.DMA,
    ),
    compiler_params=pltpu.CompilerParams(use_tc_tiling_on_sc=False),
)
def _sc_gather(user_hbm, item_hbm, fnu_tab, fni_tab, mfu_tab, mfi_tab,
               fnu_out, fni_out, mfu_out, mfi_out,
               uidx, iidx, fnu_v, fni_v, mfu_v, mfi_v, gsem, osem):
    wid = lax.axis_index("s") * _NC + lax.axis_index("c")
    base = wid * _BPW
    pltpu.sync_copy(user_hbm.at[pl.ds(base, _BPW)], uidx)
    pltpu.sync_copy(item_hbm.at[pl.ds(base, _BPW)], iidx)
    c1 = pltpu.async_copy(fnu_tab.at[uidx], fnu_v, gsem)
    c2 = pltpu.async_copy(fni_tab.at[iidx], fni_v, gsem)
    c3 = pltpu.async_copy(mfu_tab.at[uidx], mfu_v, gsem)
    c4 = pltpu.async_copy(mfi_tab.at[iidx], mfi_v, gsem)
    c1.wait()
    o1 = pltpu.async_copy(fnu_v, fnu_out.at[pl.ds(base, _BPW)], osem)
    c2.wait()
    o2 = pltpu.async_copy(fni_v, fni_out.at[pl.ds(base, _BPW)], osem)
    c3.wait()
    o3 = pltpu.async_copy(mfu_v, mfu_out.at[pl.ds(base, _BPW)], osem)
    c4.wait()
    o4 = pltpu.async_copy(mfi_v, mfi_out.at[pl.ds(base, _BPW)], osem)
    o1.wait()
    o2.wait()
    o3.wait()
    o4.wait()


def _mlp_body(fnu_ref, fni_ref, mfu_ref, mfi_ref, w1u_ref, w1i_ref, b1_ref,
              w2_ref, b2_ref, w3_ref, b3_ref, womf_ref, woh_ref, bo_ref,
              out_ref):
    f32 = jnp.float32
    h = jnp.dot(fnu_ref[...], w1u_ref[...], preferred_element_type=f32)
    h += jnp.dot(fni_ref[...], w1i_ref[...], preferred_element_type=f32)
    h = jnp.maximum(h + b1_ref[...], 0.0)
    h = jnp.maximum(
        jnp.dot(h, w2_ref[...], preferred_element_type=f32) + b2_ref[...], 0.0)
    h = jnp.maximum(
        jnp.dot(h, w3_ref[...], preferred_element_type=f32) + b3_ref[...], 0.0)
    r = jnp.dot(mfu_ref[...] * mfi_ref[...], womf_ref[...],
                preferred_element_type=f32)
    r += jnp.dot(h, woh_ref[...], preferred_element_type=f32)
    out_ref[...] = r[:, 0] + bo_ref[0, 0]


def _mlp(fnu, fni, mfu, mfi, W1, b1, W2, b2, W3, b3, Wo, bo):
    blk = 2048
    grid = _CB // blk

    def _w(shape):
        return pl.BlockSpec(shape, lambda i: (0, 0))

    return pl.pallas_call(
        _mlp_body,
        grid=(grid,),
        in_specs=[
            pl.BlockSpec((blk, _FN), lambda i: (i, 0)),
            pl.BlockSpec((blk, _FN), lambda i: (i, 0)),
            pl.BlockSpec((blk, _MF), lambda i: (i, 0)),
            pl.BlockSpec((blk, _MF), lambda i: (i, 0)),
            _w((_FN, 64)), _w((_FN, 64)), _w((1, 64)),
            _w((64, 32)), _w((1, 32)),
            _w((32, 16)), _w((1, 16)),
            _w((_MF, 1)), _w((16, 1)), _w((1, 1)),
        ],
        out_specs=pl.BlockSpec((blk,), lambda i: (i,)),
        out_shape=jax.ShapeDtypeStruct((_CB,), jnp.float32),
    )(fnu, fni, mfu, mfi,
      W1[:_FN], W1[_FN:], b1.reshape(1, 64),
      W2, b2.reshape(1, 32),
      W3, b3.reshape(1, 16),
      Wo[:_MF], Wo[_MF:], bo.reshape(1, 1))


def kernel(user, item, mf_emb_user, mf_emb_item, fn_emb_user, fn_emb_item,
           W1, b1, W2, b2, W3, b3, Wo, bo):
    user = user.astype(jnp.int32)
    item = item.astype(jnp.int32)
    gathered = [
        _sc_gather(
            lax.slice_in_dim(user, c * _CB, (c + 1) * _CB),
            lax.slice_in_dim(item, c * _CB, (c + 1) * _CB),
            fn_emb_user, fn_emb_item, mf_emb_user, mf_emb_item)
        for c in range(_NCHUNK)
    ]
    outs = [_mlp(fnu, fni, mfu, mfi, W1, b1, W2, b2, W3, b3, Wo, bo)
            for (fnu, fni, mfu, mfi) in gathered]
    return jnp.concatenate(outs)
